# Initial kernel scaffold; baseline (speedup 1.0000x reference)
import jax
import jax.numpy as jnp
from jax.experimental import pallas as pl


def kernel(x, pos_table):
    raise NotImplementedError("write your pallas kernel here")


# speedup vs baseline: 2.8465x; 2.8465x over previous
import jax
import jax.numpy as jnp
from jax.experimental import pallas as pl


def _add_body(x_ref, t_ref, o_ref):
    o_ref[...] = x_ref[...] + t_ref[...]


def kernel(x, pos_table):
    N, S, D = x.shape
    BS = 512
    return pl.pallas_call(
        _add_body,
        grid=(S // BS, N),
        in_specs=[
            pl.BlockSpec((1, BS, D), lambda j, n: (n, j, 0)),
            pl.BlockSpec((BS, D), lambda j, n: (j, 0)),
        ],
        out_specs=pl.BlockSpec((1, BS, D), lambda j, n: (n, j, 0)),
        out_shape=jax.ShapeDtypeStruct((N, S, D), x.dtype),
    )(x, pos_table)
